# baseline (speedup 1.0000x reference)
import functools
import jax
import jax.numpy as jnp
from jax import lax
from jax.experimental import pallas as pl
from jax.experimental.pallas import tpu as pltpu
from jax.experimental.pallas import tpu_sc as plsc

N_NODES = 10000
N_EDGES = 320000
D = 128
DF = 16
CHUNK = 128
SCH = 128
N_SOPS = N_EDGES // SCH
NC = 2
NS = 16
NW = NC * NS
SOPS_BASE = N_SOPS // NW
SOPS_REM = N_SOPS % NW
N_OUTER = (SOPS_BASE + 2) // 2
STRIPE = 624
TAIL = N_NODES - STRIPE * NS


def _sc_scatter_body(zeros2_hbm, zeros1_hbm, edge_hbm, dest3_hbm, psums, pcnts,
                     acc_s, idx_v, rows_v, hist_v, isem0, isem1, rsem0, rsem1,
                     ssem0, ssem1):
  core = lax.axis_index("c")
  sid = lax.axis_index("s")
  wid = sid * NC + core
  isems = (isem0, isem1)
  rsems = (rsem0, rsem1)
  ssems = (ssem0, ssem1)

  base = sid * STRIPE
  pltpu.sync_copy(zeros2_hbm.at[pl.ds(0, STRIPE)], acc_s.at[pl.ds(base, STRIPE)])

  @pl.when(sid == NS - 1)
  def _zero_tail():
    pltpu.sync_copy(zeros2_hbm.at[pl.ds(0, TAIL)],
                    acc_s.at[pl.ds(STRIPE * NS, TAIL)])

  pltpu.sync_copy(zeros1_hbm.at[0], hist_v)
  plsc.subcore_barrier()

  ones16 = jnp.ones((16,), jnp.float32)
  nsops = SOPS_BASE + jnp.where(wid < SOPS_REM, 1, 0)
  s0 = wid * SOPS_BASE + jnp.minimum(wid, SOPS_REM)

  def start_loads(b, j):
    s = s0 + j
    pltpu.async_copy(dest3_hbm.at[s], idx_v.at[b], isems[b])
    pltpu.async_copy(edge_hbm.at[pl.ds(s * SCH, SCH)], rows_v.at[b], rsems[b])

  for b in range(2):
    start_loads(b, b)

  def slot(j, b):
    @pl.when(j < nsops)
    def _do():
      pltpu.make_async_copy(dest3_hbm.at[0], idx_v.at[b], isems[b]).wait()
      pltpu.make_async_copy(edge_hbm.at[pl.ds(0, SCH)], rows_v.at[b],
                            rsems[b]).wait()
      for k in range(SCH // CHUNK):
        pltpu.async_copy(rows_v.at[b, pl.ds(k * CHUNK, CHUNK)],
                         acc_s.at[idx_v.at[b, k]], ssems[b], add=True)
        for i in range(CHUNK // 16):
          iv = idx_v[b, k, pl.ds(i * 16, 16)]
          plsc.addupdate_scatter(hist_v, [iv], ones16)
        pltpu.make_async_copy(rows_v.at[b, pl.ds(k * CHUNK, CHUNK)],
                              acc_s.at[idx_v.at[b, k]], ssems[b]).wait()

      @pl.when(j + 2 < nsops)
      def _next():
        start_loads(b, j + 2)

  def outer(g, _):
    slot(2 * g, 0)
    slot(2 * g + 1, 1)
    return 0
  lax.fori_loop(0, N_OUTER + 0 * wid, outer, 0)
  plsc.subcore_barrier()

  pltpu.sync_copy(acc_s.at[pl.ds(base, STRIPE)],
                  psums.at[core, pl.ds(base, STRIPE)])

  @pl.when(sid == NS - 1)
  def _dump_tail():
    tb = STRIPE * NS
    pltpu.sync_copy(acc_s.at[pl.ds(tb, TAIL)], psums.at[core, pl.ds(tb, TAIL)])

  pltpu.sync_copy(hist_v, pcnts.at[wid, 0])


_sc_scatter = functools.partial(
    pl.kernel,
    out_type=[
        jax.ShapeDtypeStruct((NC, N_NODES, D), jnp.float32),
        jax.ShapeDtypeStruct((NW, 1, N_NODES), jnp.float32),
    ],
    mesh=plsc.VectorSubcoreMesh(core_axis_name="c", subcore_axis_name="s"),
    scratch_types=[
        pltpu.VMEM_SHARED((N_NODES, D), jnp.float32),
        pltpu.VMEM((2, SCH // CHUNK, CHUNK), jnp.int32),
        pltpu.VMEM((2, SCH, D), jnp.float32),
        pltpu.VMEM((N_NODES,), jnp.float32),
        pltpu.SemaphoreType.DMA,
        pltpu.SemaphoreType.DMA,
        pltpu.SemaphoreType.DMA,
        pltpu.SemaphoreType.DMA,
        pltpu.SemaphoreType.DMA,
        pltpu.SemaphoreType.DMA,
    ],
    compiler_params=pltpu.CompilerParams(needs_layout_passes=False),
)(_sc_scatter_body)


def _tc_mlp_body(x_b, f_b, ps_b, pc_b, w0x, w0m, w0f, b0, w1, b1, w2, b2, o_b):
  s = ps_b[0] + ps_b[1]
  c = jnp.sum(pc_b[:, 0, :], axis=0)[:, None]
  mean = s / jnp.maximum(c, 1.0)
  h = (jnp.dot(x_b[...], w0x[...], preferred_element_type=jnp.float32)
       + jnp.dot(mean, w0m[...], preferred_element_type=jnp.float32)
       + jnp.dot(f_b[...], w0f[...], preferred_element_type=jnp.float32)
       + b0[...])
  h = h * jax.nn.sigmoid(h)
  h = jnp.dot(h, w1[...], preferred_element_type=jnp.float32) + b1[...]
  h = h * jax.nn.sigmoid(h)
  o_b[...] = jnp.dot(h, w2[...], preferred_element_type=jnp.float32) + b2[...]


_tc_mlp = pl.pallas_call(
    _tc_mlp_body,
    out_shape=jax.ShapeDtypeStruct((N_NODES, D), jnp.float32),
)


@jax.jit
def kernel(x, dest, edge_attr, f, W0, b0, W1, b1, W2, b2):
  dest3 = dest.astype(jnp.int32).reshape(N_SOPS, SCH // CHUNK, CHUNK)
  zeros2 = jnp.zeros((STRIPE + 8, D), jnp.float32)
  zeros1 = jnp.zeros((1, N_NODES), jnp.float32)
  psums, pcnts = _sc_scatter(zeros2, zeros1, edge_attr, dest3)
  w0x = W0[:D]
  w0m = W0[D:2 * D]
  w0f = W0[2 * D:]
  return _tc_mlp(x, f, psums, pcnts, w0x, w0m, w0f, b0.reshape(1, D),
                 W1, b1.reshape(1, D), W2, b2.reshape(1, D))


# speedup vs baseline: 1.0383x; 1.0383x over previous
import functools
import jax
import jax.numpy as jnp
from jax import lax
from jax.experimental import pallas as pl
from jax.experimental.pallas import tpu as pltpu
from jax.experimental.pallas import tpu_sc as plsc

N_NODES = 10000
N_EDGES = 320000
D = 128
DF = 16
CHUNK = 128
SCH = 128
N_SOPS = N_EDGES // SCH
NC = 2
NS = 16
NW = NC * NS
SOPS_BASE = N_SOPS // NW
SOPS_REM = N_SOPS % NW
N_OUTER = (SOPS_BASE + 2) // 2
STRIPE = 624
TAIL = N_NODES - STRIPE * NS


def _sc_scatter_body(edge_hbm, dest3_hbm, psums, pcnts,
                     acc_s, idx_v, rows_v, hist_v, isem0, isem1, rsem0, rsem1,
                     ssem0, ssem1):
  core = lax.axis_index("c")
  sid = lax.axis_index("s")
  wid = sid * NC + core
  isems = (isem0, isem1)
  rsems = (rsem0, rsem1)
  ssems = (ssem0, ssem1)

  zero16f = jnp.zeros((16,), jnp.float32)
  t0 = 0 * wid

  def zrow(i, _):
    for j2 in range(D // 16):
      rows_v[0, i, pl.ds(j2 * 16, 16)] = zero16f
    return 0
  lax.fori_loop(t0, SCH, zrow, 0)

  def zhist(i, _):
    hist_v[pl.ds(i * 16, 16)] = zero16f
    return 0
  lax.fori_loop(t0, N_NODES // 16, zhist, 0)

  base = sid * STRIPE

  def zblk(i, _):
    pltpu.sync_copy(rows_v.at[0], acc_s.at[pl.ds(base + i * SCH, SCH)])
    return 0
  lax.fori_loop(t0, STRIPE // SCH, zblk, 0)
  pltpu.sync_copy(rows_v.at[0, pl.ds(0, STRIPE - (STRIPE // SCH) * SCH)],
                  acc_s.at[pl.ds(base + (STRIPE // SCH) * SCH,
                                 STRIPE - (STRIPE // SCH) * SCH)])

  @pl.when(sid == NS - 1)
  def _zero_tail():
    pltpu.sync_copy(rows_v.at[0, pl.ds(0, TAIL)],
                    acc_s.at[pl.ds(STRIPE * NS, TAIL)])

  plsc.subcore_barrier()

  ones16 = jnp.ones((16,), jnp.float32)
  nsops = SOPS_BASE + jnp.where(wid < SOPS_REM, 1, 0)
  s0 = wid * SOPS_BASE + jnp.minimum(wid, SOPS_REM)

  def start_loads(b, j):
    s = s0 + j
    pltpu.async_copy(dest3_hbm.at[s], idx_v.at[b], isems[b])
    pltpu.async_copy(edge_hbm.at[pl.ds(s * SCH, SCH)], rows_v.at[b], rsems[b])

  for b in range(2):
    start_loads(b, b)

  def slot(j, b):
    @pl.when(j < nsops)
    def _do():
      pltpu.make_async_copy(dest3_hbm.at[0], idx_v.at[b], isems[b]).wait()
      pltpu.make_async_copy(edge_hbm.at[pl.ds(0, SCH)], rows_v.at[b],
                            rsems[b]).wait()
      for k in range(SCH // CHUNK):
        pltpu.async_copy(rows_v.at[b, pl.ds(k * CHUNK, CHUNK)],
                         acc_s.at[idx_v.at[b, k]], ssems[b], add=True)
        for i in range(CHUNK // 16):
          iv = idx_v[b, k, pl.ds(i * 16, 16)]
          plsc.addupdate_scatter(hist_v, [iv], ones16)
        pltpu.make_async_copy(rows_v.at[b, pl.ds(k * CHUNK, CHUNK)],
                              acc_s.at[idx_v.at[b, k]], ssems[b]).wait()

      @pl.when(j + 2 < nsops)
      def _next():
        start_loads(b, j + 2)

  def outer(g, _):
    slot(2 * g, 0)
    slot(2 * g + 1, 1)
    return 0
  lax.fori_loop(0, N_OUTER + 0 * wid, outer, 0)
  plsc.subcore_barrier()

  pltpu.sync_copy(acc_s.at[pl.ds(base, STRIPE)],
                  psums.at[core, pl.ds(base, STRIPE)])

  @pl.when(sid == NS - 1)
  def _dump_tail():
    tb = STRIPE * NS
    pltpu.sync_copy(acc_s.at[pl.ds(tb, TAIL)], psums.at[core, pl.ds(tb, TAIL)])

  pltpu.sync_copy(hist_v, pcnts.at[wid, 0])


_sc_scatter = functools.partial(
    pl.kernel,
    out_type=[
        jax.ShapeDtypeStruct((NC, N_NODES, D), jnp.float32),
        jax.ShapeDtypeStruct((NW, 1, N_NODES), jnp.float32),
    ],
    mesh=plsc.VectorSubcoreMesh(core_axis_name="c", subcore_axis_name="s"),
    scratch_types=[
        pltpu.VMEM_SHARED((N_NODES, D), jnp.float32),
        pltpu.VMEM((2, SCH // CHUNK, CHUNK), jnp.int32),
        pltpu.VMEM((2, SCH, D), jnp.float32),
        pltpu.VMEM((N_NODES,), jnp.float32),
        pltpu.SemaphoreType.DMA,
        pltpu.SemaphoreType.DMA,
        pltpu.SemaphoreType.DMA,
        pltpu.SemaphoreType.DMA,
        pltpu.SemaphoreType.DMA,
        pltpu.SemaphoreType.DMA,
    ],
    compiler_params=pltpu.CompilerParams(needs_layout_passes=False),
)(_sc_scatter_body)


def _tc_mlp_body(x_b, f_b, ps_b, pc_b, w0x, w0m, w0f, b0, w1, b1, w2, b2, o_b):
  s = ps_b[0] + ps_b[1]
  c = jnp.sum(pc_b[:, 0, :], axis=0)[:, None]
  mean = s / jnp.maximum(c, 1.0)
  h = (jnp.dot(x_b[...], w0x[...], preferred_element_type=jnp.float32)
       + jnp.dot(mean, w0m[...], preferred_element_type=jnp.float32)
       + jnp.dot(f_b[...], w0f[...], preferred_element_type=jnp.float32)
       + b0[...])
  h = h * jax.nn.sigmoid(h)
  h = jnp.dot(h, w1[...], preferred_element_type=jnp.float32) + b1[...]
  h = h * jax.nn.sigmoid(h)
  o_b[...] = jnp.dot(h, w2[...], preferred_element_type=jnp.float32) + b2[...]


_tc_mlp = pl.pallas_call(
    _tc_mlp_body,
    out_shape=jax.ShapeDtypeStruct((N_NODES, D), jnp.float32),
)


@jax.jit
def kernel(x, dest, edge_attr, f, W0, b0, W1, b1, W2, b2):
  dest3 = dest.astype(jnp.int32).reshape(N_SOPS, SCH // CHUNK, CHUNK)
  psums, pcnts = _sc_scatter(edge_attr, dest3)
  w0x = W0[:D]
  w0m = W0[D:2 * D]
  w0f = W0[2 * D:]
  return _tc_mlp(x, f, psums, pcnts, w0x, w0m, w0f, b0.reshape(1, D),
                 W1, b1.reshape(1, D), W2, b2.reshape(1, D))


# speedup vs baseline: 1.0403x; 1.0019x over previous
import functools
import jax
import jax.numpy as jnp
from jax import lax
from jax.experimental import pallas as pl
from jax.experimental.pallas import tpu as pltpu
from jax.experimental.pallas import tpu_sc as plsc

N_NODES = 10000
N_EDGES = 320000
D = 128
DF = 16
CHUNK = 128
SCH = 128
N_SOPS = N_EDGES // SCH
NC = 2
NS = 16
NW = NC * NS
SOPS_BASE = N_SOPS // NW
SOPS_REM = N_SOPS % NW
N_OUTER = (SOPS_BASE + 2) // 2
STRIPE = 624
TAIL = N_NODES - STRIPE * NS


def _sc_scatter_body(edge_hbm, dest3_hbm, psums, pcnts,
                     acc_s, idx_v, rows_v, hist_v, isem0, isem1, rsem0, rsem1,
                     ssem0, ssem1):
  core = lax.axis_index("c")
  sid = lax.axis_index("s")
  wid = sid * NC + core
  isems = (isem0, isem1)
  rsems = (rsem0, rsem1)
  ssems = (ssem0, ssem1)

  zero16f = jnp.zeros((16,), jnp.float32)
  t0 = 0 * wid
  nsops = SOPS_BASE + jnp.where(wid < SOPS_REM, 1, 0)
  s0 = wid * SOPS_BASE + jnp.minimum(wid, SOPS_REM)

  def start_loads(b, j):
    s = s0 + j
    pltpu.async_copy(dest3_hbm.at[s], idx_v.at[b], isems[b])
    pltpu.async_copy(edge_hbm.at[pl.ds(s * SCH, SCH)], rows_v.at[b], rsems[b])

  start_loads(1, 1)

  def zrow(i, _):
    for j2 in range(D // 16):
      rows_v[0, i, pl.ds(j2 * 16, 16)] = zero16f
    return 0
  lax.fori_loop(t0, SCH, zrow, 0)

  def zhist(i, _):
    hist_v[pl.ds(i * 16, 16)] = zero16f
    return 0
  lax.fori_loop(t0, N_NODES // 16, zhist, 0)

  base = sid * STRIPE

  def zblk(i, _):
    pltpu.sync_copy(rows_v.at[0], acc_s.at[pl.ds(base + i * SCH, SCH)])
    return 0
  lax.fori_loop(t0, STRIPE // SCH, zblk, 0)
  pltpu.sync_copy(rows_v.at[0, pl.ds(0, STRIPE - (STRIPE // SCH) * SCH)],
                  acc_s.at[pl.ds(base + (STRIPE // SCH) * SCH,
                                 STRIPE - (STRIPE // SCH) * SCH)])

  @pl.when(sid == NS - 1)
  def _zero_tail():
    pltpu.sync_copy(rows_v.at[0, pl.ds(0, TAIL)],
                    acc_s.at[pl.ds(STRIPE * NS, TAIL)])

  start_loads(0, 0)
  plsc.subcore_barrier()

  ones16 = jnp.ones((16,), jnp.float32)

  def slot(j, b):
    @pl.when(j < nsops)
    def _do():
      pltpu.make_async_copy(dest3_hbm.at[0], idx_v.at[b], isems[b]).wait()
      pltpu.make_async_copy(edge_hbm.at[pl.ds(0, SCH)], rows_v.at[b],
                            rsems[b]).wait()
      for k in range(SCH // CHUNK):
        pltpu.async_copy(rows_v.at[b, pl.ds(k * CHUNK, CHUNK)],
                         acc_s.at[idx_v.at[b, k]], ssems[b], add=True)
        for i in range(CHUNK // 16):
          iv = idx_v[b, k, pl.ds(i * 16, 16)]
          plsc.addupdate_scatter(hist_v, [iv], ones16)
        pltpu.make_async_copy(rows_v.at[b, pl.ds(k * CHUNK, CHUNK)],
                              acc_s.at[idx_v.at[b, k]], ssems[b]).wait()

      @pl.when(j + 2 < nsops)
      def _next():
        start_loads(b, j + 2)

  def outer(g, _):
    slot(2 * g, 0)
    slot(2 * g + 1, 1)
    return 0
  lax.fori_loop(0, N_OUTER + 0 * wid, outer, 0)
  plsc.subcore_barrier()

  pltpu.sync_copy(acc_s.at[pl.ds(base, STRIPE)],
                  psums.at[core, pl.ds(base, STRIPE)])

  @pl.when(sid == NS - 1)
  def _dump_tail():
    tb = STRIPE * NS
    pltpu.sync_copy(acc_s.at[pl.ds(tb, TAIL)], psums.at[core, pl.ds(tb, TAIL)])

  pltpu.sync_copy(hist_v, pcnts.at[wid, 0])


_sc_scatter = functools.partial(
    pl.kernel,
    out_type=[
        jax.ShapeDtypeStruct((NC, N_NODES, D), jnp.float32),
        jax.ShapeDtypeStruct((NW, 1, N_NODES), jnp.float32),
    ],
    mesh=plsc.VectorSubcoreMesh(core_axis_name="c", subcore_axis_name="s"),
    scratch_types=[
        pltpu.VMEM_SHARED((N_NODES, D), jnp.float32),
        pltpu.VMEM((2, SCH // CHUNK, CHUNK), jnp.int32),
        pltpu.VMEM((2, SCH, D), jnp.float32),
        pltpu.VMEM((N_NODES,), jnp.float32),
        pltpu.SemaphoreType.DMA,
        pltpu.SemaphoreType.DMA,
        pltpu.SemaphoreType.DMA,
        pltpu.SemaphoreType.DMA,
        pltpu.SemaphoreType.DMA,
        pltpu.SemaphoreType.DMA,
    ],
    compiler_params=pltpu.CompilerParams(needs_layout_passes=False),
)(_sc_scatter_body)


def _tc_mlp_body(x_b, f_b, ps_b, pc_b, w0x, w0m, w0f, b0, w1, b1, w2, b2, o_b):
  s = ps_b[0] + ps_b[1]
  c = jnp.sum(pc_b[:, 0, :], axis=0)[:, None]
  mean = s / jnp.maximum(c, 1.0)
  h = (jnp.dot(x_b[...], w0x[...], preferred_element_type=jnp.float32)
       + jnp.dot(mean, w0m[...], preferred_element_type=jnp.float32)
       + jnp.dot(f_b[...], w0f[...], preferred_element_type=jnp.float32)
       + b0[...])
  h = h * jax.nn.sigmoid(h)
  h = jnp.dot(h, w1[...], preferred_element_type=jnp.float32) + b1[...]
  h = h * jax.nn.sigmoid(h)
  o_b[...] = jnp.dot(h, w2[...], preferred_element_type=jnp.float32) + b2[...]


_tc_mlp = pl.pallas_call(
    _tc_mlp_body,
    out_shape=jax.ShapeDtypeStruct((N_NODES, D), jnp.float32),
)


@jax.jit
def kernel(x, dest, edge_attr, f, W0, b0, W1, b1, W2, b2):
  dest3 = dest.astype(jnp.int32).reshape(N_SOPS, SCH // CHUNK, CHUNK)
  psums, pcnts = _sc_scatter(edge_attr, dest3)
  w0x = W0[:D]
  w0m = W0[D:2 * D]
  w0f = W0[2 * D:]
  return _tc_mlp(x, f, psums, pcnts, w0x, w0m, w0f, b0.reshape(1, D),
                 W1, b1.reshape(1, D), W2, b2.reshape(1, D))
